# baseline (speedup 1.0000x reference)
import functools

import jax
import jax.numpy as jnp
from jax import lax
from jax.experimental import pallas as pl
from jax.experimental.pallas import tpu as pltpu
from jax.experimental.pallas import tpu_sc as plsc

D = 64
DP = 128
CH = 128
K = 4
NC = 2
NS = 16
NW = NC * NS


@functools.lru_cache(maxsize=None)
def _make_gather(n_rows: int):
    n_per_w = n_rows // NW
    n_ch = n_per_w // CH
    n_gr = n_ch // K
    assert n_gr * K == n_ch
    mesh = plsc.VectorSubcoreMesh(core_axis_name="c", subcore_axis_name="s")

    @functools.partial(
        pl.kernel,
        mesh=mesh,
        out_type=jax.ShapeDtypeStruct((n_rows, D), jnp.float32),
        scratch_types=[
            pltpu.VMEM((n_ch, CH), jnp.int32),
            pltpu.VMEM((K, CH, DP), jnp.float32),
            pltpu.SemaphoreType.DMA((K,)),
            pltpu.SemaphoreType.DMA((K,)),
        ],
        compiler_params=pltpu.CompilerParams(use_tc_tiling_on_sc=False),
    )
    def gather_kernel(idx_hbm, table_hbm, out_hbm, idx_v, rows_v, gsem, osem):
        wid = lax.axis_index("s") * NC + lax.axis_index("c")
        row0 = wid * n_ch
        pltpu.sync_copy(idx_hbm.at[pl.ds(row0, n_ch)], idx_v)
        base = wid * n_per_w

        def gather(j, b):
            return pltpu.make_async_copy(
                table_hbm.at[idx_v.at[j]], rows_v.at[b], gsem.at[b])

        def write(j, b):
            return pltpu.make_async_copy(
                rows_v.at[b, :, pl.ds(0, D)],
                out_hbm.at[pl.ds(base + j * CH, CH)], osem.at[b])

        for b in range(K):
            gather(b, b).start()

        def group(g, carry):
            for b in range(K):
                j = g * K + b
                gather(j, b).wait()
                w = write(j, b)
                w.start()
                w.wait()
                gather(j + K, b).start()
            return carry

        lax.fori_loop(0, n_gr - 1, group, 0)

        for b in range(K):
            j = (n_gr - 1) * K + b
            gather(j, b).wait()
            pltpu.sync_copy(rows_v.at[b, :, pl.ds(0, D)],
                            out_hbm.at[pl.ds(base + j * CH, CH)])

    return gather_kernel


def kernel(x, table):
    B, L = x.shape
    n = B * L
    tpad = jnp.pad(table, ((0, 0), (0, DP - D)))
    idx = x.reshape(n // CH, CH).astype(jnp.int32)
    out = _make_gather(n)(idx, tpad)
    return out.reshape(B, L, D)


# speedup vs baseline: 1.0458x; 1.0458x over previous
import functools

import jax
import jax.numpy as jnp
from jax import lax
from jax.experimental import pallas as pl
from jax.experimental.pallas import tpu as pltpu
from jax.experimental.pallas import tpu_sc as plsc

D = 64
DP = 128
CH = 128
K = 4
NC = 2
NS = 16
NW = NC * NS


@functools.lru_cache(maxsize=None)
def _make_gather(n_rows: int):
    n_per_w = n_rows // NW
    n_ch = n_per_w // CH
    n_gr = n_ch // K
    assert n_gr * K == n_ch
    mesh = plsc.VectorSubcoreMesh(core_axis_name="c", subcore_axis_name="s")

    @functools.partial(
        pl.kernel,
        mesh=mesh,
        out_type=jax.ShapeDtypeStruct((n_rows, DP), jnp.float32),
        scratch_types=[
            pltpu.VMEM((n_ch, CH), jnp.int32),
            pltpu.VMEM((K, CH, DP), jnp.float32),
            pltpu.SemaphoreType.DMA((K,)),
            pltpu.SemaphoreType.DMA((K,)),
        ],
        compiler_params=pltpu.CompilerParams(use_tc_tiling_on_sc=False),
    )
    def gather_kernel(idx_hbm, table_hbm, out_hbm, idx_v, rows_v, gsem, osem):
        wid = lax.axis_index("s") * NC + lax.axis_index("c")
        row0 = wid * n_ch
        pltpu.sync_copy(idx_hbm.at[pl.ds(row0, n_ch)], idx_v)
        base = wid * n_per_w

        def gather(j, b):
            return pltpu.make_async_copy(
                table_hbm.at[idx_v.at[j]], rows_v.at[b], gsem.at[b])

        def write(j, b):
            return pltpu.make_async_copy(
                rows_v.at[b], out_hbm.at[pl.ds(base + j * CH, CH)], osem.at[b])

        for b in range(K):
            gather(b, b).start()

        def group(g, carry):
            for b in range(K):
                j = g * K + b
                gather(j, b).wait()
                w = write(j, b)
                w.start()
                w.wait()
                gather(j + K, b).start()
            return carry

        lax.fori_loop(0, n_gr - 1, group, 0)

        for b in range(K):
            j = (n_gr - 1) * K + b
            gather(j, b).wait()
            pltpu.sync_copy(rows_v.at[b], out_hbm.at[pl.ds(base + j * CH, CH)])

    return gather_kernel


def kernel(x, table):
    B, L = x.shape
    n = B * L
    tpad = jnp.pad(table, ((0, 0), (0, DP - D)))
    xl = (x.T.reshape(L // 8, 8, B // CH, CH)
          .transpose(0, 2, 1, 3).reshape(n // CH, CH).astype(jnp.int32))
    out = _make_gather(n)(xl, tpad)
    o5 = out[:, :D].reshape(L // 8, B // CH, 8, CH, D)
    return o5.transpose(1, 3, 0, 2, 4).reshape(B, L, D)


# speedup vs baseline: 1.2324x; 1.1784x over previous
import functools

import jax
import jax.numpy as jnp
from jax import lax
from jax.experimental import pallas as pl
from jax.experimental.pallas import tpu as pltpu
from jax.experimental.pallas import tpu_sc as plsc

D = 64
DP = 128
CH = 128
K = 4
NC = 2
NS = 16
NW = NC * NS


@functools.lru_cache(maxsize=None)
def _make_gather(n_rows: int):
    n_per_w = n_rows // NW
    n_ch = n_per_w // CH
    n_gr = n_ch // K
    assert n_gr * K == n_ch
    mesh = plsc.VectorSubcoreMesh(core_axis_name="c", subcore_axis_name="s")

    @functools.partial(
        pl.kernel,
        mesh=mesh,
        out_type=jax.ShapeDtypeStruct((n_rows, DP), jnp.float32),
        scratch_types=[
            pltpu.VMEM((n_ch, CH), jnp.int32),
            pltpu.VMEM((K, CH, DP), jnp.float32),
            pltpu.SemaphoreType.DMA((K,)),
            pltpu.SemaphoreType.DMA((K,)),
        ],
        compiler_params=pltpu.CompilerParams(use_tc_tiling_on_sc=False),
    )
    def gather_kernel(idx_hbm, table_hbm, out_hbm, idx_v, rows_v, gsem, osem):
        wid = lax.axis_index("s") * NC + lax.axis_index("c")
        row0 = wid * n_ch
        pltpu.sync_copy(idx_hbm.at[pl.ds(row0, n_ch)], idx_v)
        base = wid * n_per_w

        def gather(j, b):
            return pltpu.make_async_copy(
                table_hbm.at[idx_v.at[j]], rows_v.at[b], gsem.at[b])

        def write(j, b):
            return pltpu.make_async_copy(
                rows_v.at[b], out_hbm.at[pl.ds(base + j * CH, CH)], osem.at[b])

        for b in range(K):
            gather(b, b).start()

        def group(g, carry):
            for b in range(K):
                j = g * K + b
                gather(j, b).wait()
                w = write(j, b)
                w.start()
                w.wait()
                gather(j + K, b).start()
            return carry

        lax.fori_loop(0, n_gr - 1, group, 0)

        for b in range(K):
            j = (n_gr - 1) * K + b
            gather(j, b).wait()
            pltpu.sync_copy(rows_v.at[b], out_hbm.at[pl.ds(base + j * CH, CH)])

    return gather_kernel


def kernel(x, table):
    B, L = x.shape
    n = B * L
    tpad = lax.dynamic_update_slice(
        jnp.zeros((table.shape[0], DP), jnp.float32), table, (0, 0))
    idx = x.reshape(n // CH, CH).astype(jnp.int32)
    out = _make_gather(n)(idx, tpad)
    return out[:, :D].reshape(B, L, D)


# speedup vs baseline: 1.2741x; 1.0339x over previous
import functools

import jax
import jax.numpy as jnp
from jax import lax
from jax.experimental import pallas as pl
from jax.experimental.pallas import tpu as pltpu
from jax.experimental.pallas import tpu_sc as plsc

D = 64
DP = 128
CH = 128
K = 4
NC = 2
NS = 16
NW = NC * NS


@functools.lru_cache(maxsize=None)
def _make_gather(n_rows: int):
    n_per_w = n_rows // NW
    n_ch = n_per_w // CH
    n_gr = n_ch // K
    assert n_gr * K == n_ch
    mesh = plsc.VectorSubcoreMesh(core_axis_name="c", subcore_axis_name="s")

    @functools.partial(
        pl.kernel,
        mesh=mesh,
        out_type=jax.ShapeDtypeStruct((n_rows, DP), jnp.float32),
        scratch_types=[
            pltpu.VMEM((n_ch, CH), jnp.int32),
            pltpu.VMEM((K, CH, DP), jnp.float32),
            pltpu.SemaphoreType.DMA((K,)),
            pltpu.SemaphoreType.DMA((K,)),
        ],
        compiler_params=pltpu.CompilerParams(use_tc_tiling_on_sc=False),
    )
    def gather_kernel(idx_hbm, table_hbm, out_hbm, idx_v, rows_v, gsem, osem):
        wid = lax.axis_index("s") * NC + lax.axis_index("c")
        row0 = wid * n_ch
        pltpu.sync_copy(idx_hbm.at[pl.ds(row0, n_ch)], idx_v)

        def gather(j, b):
            return pltpu.make_async_copy(
                table_hbm.at[idx_v.at[j]], rows_v.at[b], gsem.at[b])

        def write(j, b):
            jg = row0 + j
            seq = (jg // 256) * 8 + jg % 8
            tok_hi = (jg // 8) % 32
            dst = (seq * 32 + tok_hi) * CH
            return pltpu.make_async_copy(
                rows_v.at[b], out_hbm.at[pl.ds(dst, CH)], osem.at[b])

        for b in range(K):
            gather(b, b).start()

        def group(g, carry):
            for b in range(K):
                j = g * K + b
                gather(j, b).wait()
                w = write(j, b)
                w.start()
                w.wait()
                gather(j + K, b).start()
            return carry

        lax.fori_loop(0, n_gr - 1, group, 0)

        for b in range(K):
            j = (n_gr - 1) * K + b
            gather(j, b).wait()
            w = write(j, b)
            w.start()
            w.wait()

    return gather_kernel


def kernel(x, table):
    B, L = x.shape
    n = B * L
    tpad = jnp.pad(table, ((0, 0), (0, DP - D)))
    xl = (x.T.reshape(L // 8, 8, B // CH, CH)
          .transpose(0, 2, 1, 3).reshape(n // CH, CH).astype(jnp.int32))
    out = _make_gather(n)(xl, tpad)
    return out[:, :D].reshape(L, B, D).transpose(1, 0, 2)


# speedup vs baseline: 1.2750x; 1.0006x over previous
import functools

import jax
import jax.numpy as jnp
from jax import lax
from jax.experimental import pallas as pl
from jax.experimental.pallas import tpu as pltpu
from jax.experimental.pallas import tpu_sc as plsc

D = 64
DP = 128
CH = 128
K = 4
NC = 2
NS = 16
NW = NC * NS


@functools.lru_cache(maxsize=None)
def _make_gather(n_rows: int, n_tok_hi: int):
    n_per_w = n_rows // NW
    n_ch = n_per_w // CH
    n_gr = n_ch // K
    assert n_gr * K == n_ch
    mesh = plsc.VectorSubcoreMesh(core_axis_name="c", subcore_axis_name="s")

    @functools.partial(
        pl.kernel,
        mesh=mesh,
        out_type=jax.ShapeDtypeStruct((n_rows, DP), jnp.float32),
        scratch_types=[
            pltpu.VMEM((n_ch, CH), jnp.int32),
            pltpu.VMEM((K, CH, DP), jnp.float32),
            pltpu.SemaphoreType.DMA((K,)),
            pltpu.SemaphoreType.DMA((K,)),
        ],
        compiler_params=pltpu.CompilerParams(use_tc_tiling_on_sc=False),
    )
    def gather_kernel(idx_hbm, table_hbm, out_hbm, idx_v, rows_v, gsem, osem):
        wid = lax.axis_index("s") * NC + lax.axis_index("c")
        row0 = wid * n_ch
        pltpu.sync_copy(idx_hbm.at[pl.ds(row0, n_ch)], idx_v)

        def gather(j, b):
            return pltpu.make_async_copy(
                table_hbm.at[idx_v.at[j]], rows_v.at[b], gsem.at[b])

        def write(j, b):
            jg = row0 + j
            seq = (jg // (8 * n_tok_hi)) * 8 + jg % 8
            tok_hi = (jg // 8) % n_tok_hi
            dst = (seq * n_tok_hi + tok_hi) * CH
            return pltpu.make_async_copy(
                rows_v.at[b], out_hbm.at[pl.ds(dst, CH)], osem.at[b])

        for b in range(K):
            gather(b, b).start()

        def group(g, carry):
            for b in range(K):
                j = g * K + b
                gather(j, b).wait()
                w = write(j, b)
                w.start()
                w.wait()
                gather(j + K, b).start()
            return carry

        lax.fori_loop(0, n_gr - 1, group, 0)

        for b in range(K):
            j = (n_gr - 1) * K + b
            gather(j, b).wait()
            w = write(j, b)
            w.start()
            w.wait()

    return gather_kernel


def kernel(x, table):
    B, L = x.shape
    n = B * L
    tpad = jnp.pad(table, ((0, 0), (0, DP - D)))
    xl = (x.T.reshape(L // 8, 8, B // CH, CH)
          .transpose(0, 2, 1, 3).reshape(n // CH, CH).astype(jnp.int32))
    out = _make_gather(n, B // CH)(xl, tpad)
    return out[:, :D].reshape(L, B, D).transpose(1, 0, 2)
